# baseline (speedup 1.0000x reference)
import numpy as np

import jax
import jax.numpy as jnp
from jax import lax
from jax.experimental import pallas as pl
from jax.experimental.pallas import tpu as pltpu
from jax.experimental.pallas import tpu_sc as plsc

B = 16384
F = 6
H = 16
L = 16
NW = 16
ROWS_PER_W = B // NW
CHUNKS = ROWS_PER_W // L
MED_RANK = B // 2
WPACK = F * H + H + H + L

LN2 = np.float32(0.6931471805599453)

_MESH = plsc.VectorSubcoreMesh(
    core_axis_name="c", subcore_axis_name="s", num_cores=1)
_PARAMS = pltpu.CompilerParams(needs_layout_passes=False)


def _body(sp, va, ma, no, sk, co, wb, out,
          sp_v, va_v, ma_v, no_v, sk_v, co_v, wb_v,
          row_v, cnt_v, out_v, shared, sem):
    sid = lax.axis_index("s")
    base = sid * ROWS_PER_W
    copies = [
        pltpu.async_copy(hbm.at[pl.ds(base, ROWS_PER_W)], vmem, sem)
        for hbm, vmem in ((sp, sp_v), (va, va_v), (ma, ma_v),
                          (no, no_v), (sk, sk_v), (co, co_v))
    ]
    copies.append(pltpu.async_copy(wb, wb_v, sem))
    for c in copies:
        c.wait()

    w1rows = [wb_v[pl.ds(j * H, H)] for j in range(F)]
    b1vec = wb_v[pl.ds(F * H, H)]
    w2vec = wb_v[pl.ds(F * H + H, H)]
    w1s = [[w1rows[j][i] for i in range(H)] for j in range(F)]
    b1s = [b1vec[i] for i in range(H)]
    w2s = [w2vec[i] for i in range(H)]
    b2s = wb_v[pl.ds(F * H + 2 * H, L)][0]

    zero = jnp.zeros((L,), jnp.float32)

    @plsc.parallel_loop(0, ROWS_PER_W, step=L, unroll=4, carry=(zero, zero))
    def chunk(base_row, carry):
        acc2, acc3 = carry
        f = [ref[pl.ds(base_row, L)]
             for ref in (sp_v, va_v, ma_v, no_v, sk_v, co_v)]
        logit = jnp.full((L,), b2s, dtype=jnp.float32)
        for i in range(H):
            h = b1s[i] + w1s[0][i] * f[0]
            for j in range(1, F):
                h = h + w1s[j][i] * f[j]
            h = jnp.maximum(h, 0.0)
            logit = logit + w2s[i] * h
        acc2 = acc2 + jnp.where(logit < -LN2, 1.0, 0.0)
        acc3 = acc3 + jnp.where(logit < LN2, 1.0, 0.0)
        return acc2, acc3

    acc2, acc3 = chunk
    c2 = jnp.sum(acc2)
    c3 = jnp.sum(acc3)
    lane = jnp.arange(L, dtype=jnp.int32)
    row_v[...] = jnp.where(lane == 0, c2, jnp.where(lane == 1, c3, 0.0))
    pltpu.sync_copy(row_v, shared.at[pl.ds(sid * L, L)])
    plsc.subcore_barrier()

    @pl.when(sid == 0)
    def _():
        pltpu.sync_copy(shared, cnt_v)
        acc = cnt_v[pl.ds(0, L)]
        for w in range(1, NW):
            acc = acc + cnt_v[pl.ds(w * L, L)]
        tot2 = jnp.sum(jnp.where(lane == 0, acc, 0.0))
        tot3 = jnp.sum(jnp.where(lane == 1, acc, 0.0))
        k = (1.0 + jnp.where(tot2 < float(MED_RANK), 1.0, 0.0)
                 + jnp.where(tot3 < float(MED_RANK), 1.0, 0.0))
        out_v[...] = jnp.full((L,), k, dtype=jnp.float32)
        pltpu.sync_copy(out_v, out)


_selector = pl.kernel(
    _body,
    out_type=jax.ShapeDtypeStruct((L,), jnp.float32),
    mesh=_MESH,
    scratch_types=[
        pltpu.VMEM((ROWS_PER_W,), jnp.float32),
        pltpu.VMEM((ROWS_PER_W,), jnp.float32),
        pltpu.VMEM((ROWS_PER_W,), jnp.float32),
        pltpu.VMEM((ROWS_PER_W,), jnp.float32),
        pltpu.VMEM((ROWS_PER_W,), jnp.float32),
        pltpu.VMEM((ROWS_PER_W,), jnp.float32),
        pltpu.VMEM((WPACK,), jnp.float32),
        pltpu.VMEM((L,), jnp.float32),
        pltpu.VMEM((NW * L,), jnp.float32),
        pltpu.VMEM((L,), jnp.float32),
        pltpu.VMEM_SHARED((NW * L,), jnp.float32),
        pltpu.SemaphoreType.DMA,
    ],
    compiler_params=_PARAMS,
)


def kernel(x, sparsity, variance, magnitude, norm, skewness, concentration,
           W1, b1, W2, b2):
    del x
    wb = jnp.concatenate([W1.reshape(F * H), b1, W2.reshape(H),
                          jnp.broadcast_to(b2, (L,))])
    out16 = _selector(sparsity, variance, magnitude, norm, skewness,
                      concentration, wb)
    return out16[0]


# speedup vs baseline: 1.0789x; 1.0789x over previous
import numpy as np

import jax
import jax.numpy as jnp
from jax import lax
from jax.experimental import pallas as pl
from jax.experimental.pallas import tpu as pltpu
from jax.experimental.pallas import tpu_sc as plsc

B = 16384
F = 6
H = 16
L = 16
NC = 2
NS = 16
ROWS_PER_W = B // (NC * NS)
CHUNKS = ROWS_PER_W // L
MED_RANK = B // 2
WPACK = F * H + H + H + L

LN2 = np.float32(0.6931471805599453)

_MESH = plsc.VectorSubcoreMesh(
    core_axis_name="c", subcore_axis_name="s", num_cores=NC)
_PARAMS = pltpu.CompilerParams(needs_layout_passes=False)


def _body(sp, va, ma, no, sk, co, wb, out,
          sp_v, va_v, ma_v, no_v, sk_v, co_v, wb_v,
          row_v, cnt_v, out_v, shared, sem):
    cid = lax.axis_index("c")
    sid = lax.axis_index("s")
    base = (cid * NS + sid) * ROWS_PER_W
    copies = [
        pltpu.async_copy(hbm.at[pl.ds(base, ROWS_PER_W)], vmem, sem)
        for hbm, vmem in ((sp, sp_v), (va, va_v), (ma, ma_v),
                          (no, no_v), (sk, sk_v), (co, co_v))
    ]
    copies.append(pltpu.async_copy(wb, wb_v, sem))
    for c in copies:
        c.wait()

    w1rows = [wb_v[pl.ds(j * H, H)] for j in range(F)]
    b1vec = wb_v[pl.ds(F * H, H)]
    w2vec = wb_v[pl.ds(F * H + H, H)]
    w1s = [[w1rows[j][i] for i in range(H)] for j in range(F)]
    b1s = [b1vec[i] for i in range(H)]
    w2s = [w2vec[i] for i in range(H)]
    b2s = wb_v[pl.ds(F * H + 2 * H, L)][0]

    def chunk(c, carry):
        acc2, acc3 = carry
        f = [ref[pl.ds(c * L, L)]
             for ref in (sp_v, va_v, ma_v, no_v, sk_v, co_v)]
        logit = jnp.full((L,), b2s, dtype=jnp.float32)
        for i in range(H):
            h = b1s[i] + w1s[0][i] * f[0]
            for j in range(1, F):
                h = h + w1s[j][i] * f[j]
            h = jnp.maximum(h, 0.0)
            logit = logit + w2s[i] * h
        acc2 = acc2 + jnp.where(logit < -LN2, 1.0, 0.0)
        acc3 = acc3 + jnp.where(logit < LN2, 1.0, 0.0)
        return acc2, acc3

    zero = jnp.zeros((L,), jnp.float32)
    acc2, acc3 = lax.fori_loop(0, CHUNKS, chunk, (zero, zero))
    c2 = jnp.sum(acc2)
    c3 = jnp.sum(acc3)
    lane = jnp.arange(L, dtype=jnp.int32)
    row_v[...] = jnp.where(lane == 0, c2, jnp.where(lane == 1, c3, 0.0))
    pltpu.sync_copy(row_v, shared.at[pl.ds(sid * L, L)])
    plsc.subcore_barrier()

    @pl.when(sid == 0)
    def _():
        pltpu.sync_copy(shared, cnt_v)
        acc = cnt_v[pl.ds(0, L)]
        for w in range(1, NS):
            acc = acc + cnt_v[pl.ds(w * L, L)]
        out_v[...] = acc
        pltpu.sync_copy(out_v, out.at[pl.ds(cid * L, L)])


_selector = pl.kernel(
    _body,
    out_type=jax.ShapeDtypeStruct((NC * L,), jnp.float32),
    mesh=_MESH,
    scratch_types=[
        pltpu.VMEM((ROWS_PER_W,), jnp.float32),
        pltpu.VMEM((ROWS_PER_W,), jnp.float32),
        pltpu.VMEM((ROWS_PER_W,), jnp.float32),
        pltpu.VMEM((ROWS_PER_W,), jnp.float32),
        pltpu.VMEM((ROWS_PER_W,), jnp.float32),
        pltpu.VMEM((ROWS_PER_W,), jnp.float32),
        pltpu.VMEM((WPACK,), jnp.float32),
        pltpu.VMEM((L,), jnp.float32),
        pltpu.VMEM((NS * L,), jnp.float32),
        pltpu.VMEM((L,), jnp.float32),
        pltpu.VMEM_SHARED((NS * L,), jnp.float32),
        pltpu.SemaphoreType.DMA,
    ],
    compiler_params=_PARAMS,
)


def kernel(x, sparsity, variance, magnitude, norm, skewness, concentration,
           W1, b1, W2, b2):
    del x
    wb = jnp.concatenate([W1.reshape(F * H), b1, W2.reshape(H),
                          jnp.broadcast_to(b2, (L,))])
    cnts = _selector(sparsity, variance, magnitude, norm, skewness,
                     concentration, wb)
    tot2 = cnts[0] + cnts[L]
    tot3 = cnts[1] + cnts[L + 1]
    half = jnp.float32(MED_RANK)
    return (1.0 + (tot2 < half).astype(jnp.float32)
                + (tot3 < half).astype(jnp.float32))


# speedup vs baseline: 1.1000x; 1.0195x over previous
import numpy as np

import jax
import jax.numpy as jnp
from jax import lax
from jax.experimental import pallas as pl
from jax.experimental.pallas import tpu as pltpu
from jax.experimental.pallas import tpu_sc as plsc

B = 16384
F = 6
H = 16
L = 16
NC = 2
NS = 16
ROWS_PER_W = B // (NC * NS)
CHUNKS = ROWS_PER_W // L
MED_RANK = B // 2
WPACK = F * H + H + H + L

LN2 = np.float32(0.6931471805599453)

_MESH = plsc.VectorSubcoreMesh(
    core_axis_name="c", subcore_axis_name="s", num_cores=NC)
_PARAMS = pltpu.CompilerParams(needs_layout_passes=False)


def _body(stats, wb, out, st_v, wb_v, row_v, cnt_v, shared, sem):
    cid = lax.axis_index("c")
    sid = lax.axis_index("s")
    base = (cid * NS + sid) * ROWS_PER_W
    copies = [
        pltpu.async_copy(stats.at[pl.ds(j * B + base, ROWS_PER_W)],
                         st_v.at[pl.ds(j * ROWS_PER_W, ROWS_PER_W)], sem)
        for j in range(F)
    ]
    copies.append(pltpu.async_copy(wb, wb_v, sem))
    for c in copies:
        c.wait()

    w1rows = [wb_v[pl.ds(j * H, H)] for j in range(F)]
    b1vec = wb_v[pl.ds(F * H, H)]
    w2vec = wb_v[pl.ds(F * H + H, H)]
    w1s = [[w1rows[j][i] for i in range(H)] for j in range(F)]
    b1s = [b1vec[i] for i in range(H)]
    w2s = [w2vec[i] for i in range(H)]
    b2s = wb_v[pl.ds(F * H + 2 * H, L)][0]

    def one_vreg(row0):
        f = [st_v[pl.ds(j * ROWS_PER_W + row0, L)] for j in range(F)]
        logit = jnp.full((L,), b2s, dtype=jnp.float32)
        for i in range(H):
            h = b1s[i] + w1s[0][i] * f[0]
            for j in range(1, F):
                h = h + w1s[j][i] * f[j]
            h = jnp.maximum(h, 0.0)
            logit = logit + w2s[i] * h
        return logit

    def chunk(c, carry):
        acc2, acc3 = carry
        la = one_vreg(c * (2 * L))
        lb = one_vreg(c * (2 * L) + L)
        acc2 = (acc2 + jnp.where(la < -LN2, 1.0, 0.0)
                     + jnp.where(lb < -LN2, 1.0, 0.0))
        acc3 = (acc3 + jnp.where(la < LN2, 1.0, 0.0)
                     + jnp.where(lb < LN2, 1.0, 0.0))
        return acc2, acc3

    zero = jnp.zeros((L,), jnp.float32)
    acc2, acc3 = lax.fori_loop(0, CHUNKS // 2, chunk, (zero, zero))
    c2 = jnp.sum(acc2)
    c3 = jnp.sum(acc3)
    lane = jnp.arange(L, dtype=jnp.int32)
    row_v[...] = jnp.where(lane == 0, c2, jnp.where(lane == 1, c3, 0.0))
    pltpu.sync_copy(row_v, shared.at[pl.ds(sid * L, L)])
    plsc.subcore_barrier()

    @pl.when(sid == 0)
    def _():
        pltpu.sync_copy(shared, cnt_v)
        acc = cnt_v[pl.ds(0, L)]
        for w in range(1, NS):
            acc = acc + cnt_v[pl.ds(w * L, L)]
        row_v[...] = acc
        pltpu.sync_copy(row_v, out.at[pl.ds(cid * L, L)])


_selector = pl.kernel(
    _body,
    out_type=jax.ShapeDtypeStruct((NC * L,), jnp.float32),
    mesh=_MESH,
    scratch_types=[
        pltpu.VMEM((F * ROWS_PER_W,), jnp.float32),
        pltpu.VMEM((WPACK,), jnp.float32),
        pltpu.VMEM((L,), jnp.float32),
        pltpu.VMEM((NS * L,), jnp.float32),
        pltpu.VMEM_SHARED((NS * L,), jnp.float32),
        pltpu.SemaphoreType.DMA,
    ],
    compiler_params=_PARAMS,
)


def kernel(x, sparsity, variance, magnitude, norm, skewness, concentration,
           W1, b1, W2, b2):
    del x
    stats = jnp.concatenate([sparsity, variance, magnitude, norm,
                             skewness, concentration])
    wb = jnp.concatenate([W1.reshape(F * H), b1, W2.reshape(H),
                          jnp.broadcast_to(b2, (L,))])
    cnts = _selector(stats, wb)
    tot2 = cnts[0] + cnts[L]
    tot3 = cnts[1] + cnts[L + 1]
    half = jnp.float32(MED_RANK)
    return (1.0 + (tot2 < half).astype(jnp.float32)
                + (tot3 < half).astype(jnp.float32))


# speedup vs baseline: 1.1107x; 1.0097x over previous
import numpy as np

import jax
import jax.numpy as jnp
from jax import lax
from jax.experimental import pallas as pl
from jax.experimental.pallas import tpu as pltpu
from jax.experimental.pallas import tpu_sc as plsc

B = 16384
F = 6
H = 16
L = 16
NS = 16
ROWS_PER_W = B // NS
CHUNKS = ROWS_PER_W // L
UNROLL = 4
MED_RANK = B // 2
WPACK = F * H + H + H + L

LN2 = np.float32(0.6931471805599453)

_MESH = plsc.VectorSubcoreMesh(
    core_axis_name="c", subcore_axis_name="s", num_cores=1)
_PARAMS = pltpu.CompilerParams(needs_layout_passes=False)


def _body(stats, wb, out, st_v, wb_v, row_v, cnt_v, shared, sem):
    sid = lax.axis_index("s")
    base = sid * ROWS_PER_W
    copies = [
        pltpu.async_copy(stats.at[pl.ds(j * B + base, ROWS_PER_W)],
                         st_v.at[pl.ds(j * ROWS_PER_W, ROWS_PER_W)], sem)
        for j in range(F)
    ]
    copies.append(pltpu.async_copy(wb, wb_v, sem))
    for c in copies:
        c.wait()

    w1rows = [wb_v[pl.ds(j * H, H)] for j in range(F)]
    b1vec = wb_v[pl.ds(F * H, H)]
    w2vec = wb_v[pl.ds(F * H + H, H)]
    w1s = [[w1rows[j][i] for i in range(H)] for j in range(F)]
    b1s = [b1vec[i] for i in range(H)]
    w2s = [w2vec[i] for i in range(H)]
    b2s = wb_v[pl.ds(F * H + 2 * H, L)][0]

    def one_vreg(row0):
        f = [st_v[pl.ds(j * ROWS_PER_W + row0, L)] for j in range(F)]
        logit = jnp.full((L,), b2s, dtype=jnp.float32)
        for i in range(H):
            h = b1s[i] + w1s[0][i] * f[0]
            for j in range(1, F):
                h = h + w1s[j][i] * f[j]
            h = jnp.maximum(h, 0.0)
            logit = logit + w2s[i] * h
        return logit

    def chunk(c, carry):
        acc2, acc3 = carry
        logits = [one_vreg(c * (UNROLL * L) + u * L) for u in range(UNROLL)]
        for lg in logits:
            acc2 = acc2 + jnp.where(lg < -LN2, 1.0, 0.0)
            acc3 = acc3 + jnp.where(lg < LN2, 1.0, 0.0)
        return acc2, acc3

    zero = jnp.zeros((L,), jnp.float32)
    acc2, acc3 = lax.fori_loop(0, CHUNKS // UNROLL, chunk, (zero, zero))
    c2 = jnp.sum(acc2)
    c3 = jnp.sum(acc3)
    lane = jnp.arange(L, dtype=jnp.int32)
    row_v[...] = jnp.where(lane == 0, c2, jnp.where(lane == 1, c3, 0.0))
    pltpu.sync_copy(row_v, shared.at[pl.ds(sid * L, L)])
    plsc.subcore_barrier()

    @pl.when(sid == 0)
    def _():
        pltpu.sync_copy(shared, cnt_v)
        acc = cnt_v[pl.ds(0, L)]
        for w in range(1, NS):
            acc = acc + cnt_v[pl.ds(w * L, L)]
        tot2 = jnp.sum(jnp.where(lane == 0, acc, 0.0))
        tot3 = jnp.sum(jnp.where(lane == 1, acc, 0.0))
        k = (1.0 + jnp.where(tot2 < float(MED_RANK), 1.0, 0.0)
                 + jnp.where(tot3 < float(MED_RANK), 1.0, 0.0))
        row_v[...] = jnp.full((L,), k, dtype=jnp.float32)
        pltpu.sync_copy(row_v, out)


_selector = pl.kernel(
    _body,
    out_type=jax.ShapeDtypeStruct((L,), jnp.float32),
    mesh=_MESH,
    scratch_types=[
        pltpu.VMEM((F * ROWS_PER_W,), jnp.float32),
        pltpu.VMEM((WPACK,), jnp.float32),
        pltpu.VMEM((L,), jnp.float32),
        pltpu.VMEM((NS * L,), jnp.float32),
        pltpu.VMEM_SHARED((NS * L,), jnp.float32),
        pltpu.SemaphoreType.DMA,
    ],
    compiler_params=_PARAMS,
)


def kernel(x, sparsity, variance, magnitude, norm, skewness, concentration,
           W1, b1, W2, b2):
    del x
    stats = jnp.concatenate([sparsity, variance, magnitude, norm,
                             skewness, concentration])
    wb = jnp.concatenate([W1.reshape(F * H), b1, W2.reshape(H),
                          jnp.broadcast_to(b2, (L,))])
    out16 = _selector(stats, wb)
    return out16[0]
